# baseline (speedup 1.0000x reference)
import functools
import math

import jax
import jax.numpy as jnp
from jax import lax
from jax.experimental import pallas as pl
from jax.experimental.pallas import tpu as pltpu
from jax.experimental.pallas import tpu_sc as plsc

_MODEL_DIM = 64
_VOCAB = 1000000
_BATCH = 16384
_SCALE = math.sqrt(_MODEL_DIM)

_info = plsc.get_sparse_core_info()
_NC = _info.num_cores
_NS = _info.num_subcores
_L = _info.num_lanes
_NW = _NC * _NS
_TCOL = 128
_NCOLS = (_VOCAB + _TCOL - 1) // _TCOL
_RCOLS = 4
_RW = _RCOLS * _TCOL
_NROUND = ((_NCOLS + _NW - 1) // _NW + _RCOLS - 1) // _RCOLS
_HITCAP = 704
_ROWCAP = 32
_XCHUNK = 2048
_YROWS = _BATCH + 64
_SENTINEL = _NCOLS * _TCOL + 1

_mesh = plsc.VectorSubcoreMesh(core_axis_name="c", subcore_axis_name="s")


@functools.partial(
    pl.kernel,
    mesh=_mesh,
    compiler_params=pltpu.CompilerParams(needs_layout_passes=False),
    out_type=jax.ShapeDtypeStruct((_YROWS, _TCOL), jnp.float32),
    scratch_types=[
        pltpu.VMEM((2, _XCHUNK), jnp.int32),
        pltpu.VMEM((2, _MODEL_DIM, _RW), jnp.float32),
        pltpu.VMEM((_HITCAP + _L,), jnp.int32),
        pltpu.VMEM((_HITCAP + _L,), jnp.int32),
        pltpu.VMEM((2, _ROWCAP, _TCOL), jnp.float32),
        pltpu.VMEM((2, _ROWCAP), jnp.int32),
        pltpu.VMEM((1, _TCOL), jnp.float32),
        pltpu.SemaphoreType.DMA,
        pltpu.SemaphoreType.DMA,
        pltpu.SemaphoreType.DMA,
    ],
)
def _emb_sweep(x_hbm, wt_hbm, y_hbm, xc_v, tb, hv_v, hp_v, rows2, pos2,
               junk_v, semx, semf, sems):
    wid = lax.axis_index("s") * _NC + lax.axis_index("c")
    col_lo = wid * _NCOLS // _NW
    col_hi = (wid + 1) * _NCOLS // _NW
    lane = lax.iota(jnp.int32, _L)
    lane0 = lane == 0
    dump_row = _BATCH + wid

    def full(v):
        return jnp.full((_L,), v, jnp.int32)

    def fill_sent(i, c):
        hv_v[pl.ds(i * _L, _L)] = full(_SENTINEL)
        return c

    lax.fori_loop(0, (_HITCAP + _L) // _L, fill_sent, 0)

    first_x = pltpu.async_copy(x_hbm.at[pl.ds(0, _XCHUNK)], xc_v.at[0], semx)
    first_x.wait()

    def scan_chunk(ch, cnt):
        nxt = jnp.minimum(ch + 1, _BATCH // _XCHUNK - 1)
        cp = pltpu.async_copy(
            x_hbm.at[pl.ds(nxt * _XCHUNK, _XCHUNK)], xc_v.at[(ch + 1) % 2],
            semx)

        def scan_vec(t, cnt2):
            v = xc_v[ch % 2, pl.ds(t * _L, _L)]
            c = lax.shift_right_logical(v, 7)
            mask = (c >= col_lo) & (c < col_hi)
            pos = full(ch * _XCHUNK + t * _L) + lane
            cc = jnp.minimum(cnt2, _HITCAP)
            plsc.store_compressed(hv_v.at[pl.ds(cc, _L)], v, mask=mask)
            plsc.store_compressed(hp_v.at[pl.ds(cc, _L)], pos, mask=mask)
            return cnt2 + plsc.all_reduce_population_count(mask)[0]

        cnt = lax.fori_loop(0, _XCHUNK // _L, scan_vec, cnt)
        cp.wait()
        return cnt

    cnt = lax.fori_loop(0, _BATCH // _XCHUNK, scan_chunk, 0)
    def repad(i, c):
        s = jnp.minimum(cnt + i * _L, _HITCAP)
        vtail = hv_v[pl.ds(s, _L)]
        hv_v[pl.ds(s, _L)] = jnp.where(lane + s >= cnt, full(_SENTINEL), vtail)
        return c

    repad(0, 0)

    nlvec = (_HITCAP + _L) // _L

    def fetch(r, slot):
        start = jnp.minimum(col_lo + r * _RCOLS, _NCOLS - _RCOLS) * _TCOL
        return pltpu.async_copy(
            wt_hbm.at[:, pl.ds(pl.multiple_of(start, _TCOL), _RW)],
            tb.at[slot], semf)

    for _ in range(2 * _ROWCAP):
        pltpu.async_copy(y_hbm.at[pl.ds(_BATCH, 1)], junk_v, sems)

    fetch(0, 0)

    def sweep_round(r, carry):
        fetch(r + 1, (r + 1) % 2)
        rb = r % 2
        pltpu.make_async_copy(
            y_hbm.at[pl.ds(_BATCH, _ROWCAP)], rows2.at[rb], sems).wait()
        pltpu.make_async_copy(
            wt_hbm.at[:, pl.ds(0, _RW)], tb.at[rb], semf).wait()

        pos2[rb, pl.ds(0, _L)] = full(dump_row)
        pos2[rb, pl.ds(_L, _L)] = full(dump_row)
        lo_r = col_lo + r * _RCOLS
        hi_r = jnp.minimum(lo_r + _RCOLS, col_hi)
        off = jnp.minimum(lo_r, _NCOLS - _RCOLS) * _TCOL
        slotv = full(rb)

        def rescan(t, hw):
            hvv = hv_v[pl.ds(t * _L, _L)]
            cvv = lax.shift_right_logical(hvv, 7)
            mask = (cvv >= lo_r) & (cvv < hi_r)

            def has_hits(state):
                m, _ = state
                return plsc.all_reduce_population_count(m)[0] > 0

            def take_hit(state):
                m, hw2 = state
                l = plsc.all_reduce_ffs(m)[0]
                lv = full(l)
                sel = lane == lv
                v_l = jnp.sum(jnp.where(sel, hvv, 0))
                p_l = jnp.sum(jnp.where(sel, hp_v[pl.ds(t * _L, _L)], 0))
                hw_c = jnp.minimum(hw2, _ROWCAP - 1)
                plsc.store_scatter(pos2.at[rb], [full(hw_c)], full(p_l),
                                   mask=lane0)
                bc = full(v_l - off)
                for f16 in range(_MODEL_DIM // _L):
                    fvec = lane + f16 * _L
                    vals = plsc.load_gather(tb, [slotv, fvec, bc]) * _SCALE
                    rows2[rb, hw_c, pl.ds(f16 * _L, _L)] = vals
                return m & (lane != lv), hw2 + 1

            _, hw = lax.while_loop(has_hits, take_hit, (mask, hw))
            return hw

        lax.fori_loop(0, nlvec, rescan, 0)
        pltpu.async_copy(rows2.at[rb], y_hbm.at[pos2.at[rb]], sems)
        return carry

    lax.fori_loop(0, _NROUND, sweep_round, 0)

    pltpu.make_async_copy(
        wt_hbm.at[:, pl.ds(0, _RW)], tb.at[0], semf).wait()
    pltpu.make_async_copy(
        y_hbm.at[pl.ds(_BATCH, _ROWCAP)], rows2.at[0], sems).wait()
    pltpu.make_async_copy(
        y_hbm.at[pl.ds(_BATCH, _ROWCAP)], rows2.at[1], sems).wait()


def kernel(x, W):
    y = _emb_sweep(x, W.T)
    return y[:_BATCH, :_MODEL_DIM]


# speedup vs baseline: 1.0174x; 1.0174x over previous
import functools
import math

import jax
import jax.numpy as jnp
from jax import lax
from jax.experimental import pallas as pl
from jax.experimental.pallas import tpu as pltpu
from jax.experimental.pallas import tpu_sc as plsc

_MODEL_DIM = 64
_VOCAB = 1000000
_BATCH = 16384
_SCALE = math.sqrt(_MODEL_DIM)

_info = plsc.get_sparse_core_info()
_NC = _info.num_cores
_NS = _info.num_subcores
_L = _info.num_lanes
_NW = _NC * _NS
_TCOL = 128
_NCOLS = (_VOCAB + _TCOL - 1) // _TCOL
_RCOLS = 4
_RW = _RCOLS * _TCOL
_NROUND = ((_NCOLS + _NW - 1) // _NW + _RCOLS - 1) // _RCOLS
_HITCAP = 704
_ROWCAP = 32
_XCHUNK = 2048
_YROWS = _BATCH + 64
_SENTINEL = _NCOLS * _TCOL + 1

_mesh = plsc.VectorSubcoreMesh(core_axis_name="c", subcore_axis_name="s")


@functools.partial(
    pl.kernel,
    mesh=_mesh,
    compiler_params=pltpu.CompilerParams(needs_layout_passes=False),
    out_type=jax.ShapeDtypeStruct((_YROWS, _TCOL), jnp.float32),
    scratch_types=[
        pltpu.VMEM((2, _XCHUNK), jnp.int32),
        pltpu.VMEM((3 * _RCOLS, _MODEL_DIM, _TCOL), jnp.float32),
        pltpu.VMEM((_HITCAP + _L,), jnp.int32),
        pltpu.VMEM((_HITCAP + _L,), jnp.int32),
        pltpu.VMEM((2, _ROWCAP, _TCOL), jnp.float32),
        pltpu.VMEM((2, _ROWCAP), jnp.int32),
        pltpu.VMEM((1, _TCOL), jnp.float32),
        pltpu.SemaphoreType.DMA,
        pltpu.SemaphoreType.DMA,
        pltpu.SemaphoreType.DMA,
    ],
)
def _emb_sweep(x_hbm, wt_hbm, y_hbm, xc_v, tb, hv_v, hp_v, rows2, pos2,
               junk_v, semx, semf, sems):
    wid = lax.axis_index("s") * _NC + lax.axis_index("c")
    col_lo = wid * _NCOLS // _NW
    col_hi = (wid + 1) * _NCOLS // _NW
    lane = lax.iota(jnp.int32, _L)
    lane0 = lane == 0
    dump_row = _BATCH + wid

    def full(v):
        return jnp.full((_L,), v, jnp.int32)

    def fill_sent(i, c):
        hv_v[pl.ds(i * _L, _L)] = full(_SENTINEL)
        return c

    lax.fori_loop(0, (_HITCAP + _L) // _L, fill_sent, 0)

    def fetch(r):
        base = (r % 3) * _RCOLS
        for j in range(_RCOLS):
            q = jnp.minimum(col_lo + r * _RCOLS + j, _NCOLS - 1) * _TCOL
            pltpu.async_copy(
                wt_hbm.at[:, pl.ds(pl.multiple_of(q, _TCOL), _TCOL)],
                tb.at[base + j], semf)

    for _ in range(2 * _ROWCAP):
        pltpu.async_copy(y_hbm.at[pl.ds(_BATCH, 1)], junk_v, sems)
    fetch(0)
    fetch(1)

    first_x = pltpu.async_copy(x_hbm.at[pl.ds(0, _XCHUNK)], xc_v.at[0], semx)
    first_x.wait()

    def scan_chunk(ch, cnt):
        nxt = jnp.minimum(ch + 1, _BATCH // _XCHUNK - 1)
        cp = pltpu.async_copy(
            x_hbm.at[pl.ds(nxt * _XCHUNK, _XCHUNK)], xc_v.at[(ch + 1) % 2],
            semx)

        def scan_vec(t, cnt2):
            v = xc_v[ch % 2, pl.ds(t * _L, _L)]
            c = lax.shift_right_logical(v, 7)
            mask = (c >= col_lo) & (c < col_hi)
            pos = full(ch * _XCHUNK + t * _L) + lane
            cc = jnp.minimum(cnt2, _HITCAP)
            plsc.store_compressed(hv_v.at[pl.ds(cc, _L)], v, mask=mask)
            plsc.store_compressed(hp_v.at[pl.ds(cc, _L)], pos, mask=mask)
            return cnt2 + plsc.all_reduce_population_count(mask)[0]

        cnt = lax.fori_loop(0, _XCHUNK // _L, scan_vec, cnt)
        cp.wait()
        return cnt

    cnt = lax.fori_loop(0, _BATCH // _XCHUNK, scan_chunk, 0)
    def repad(i, c):
        s = jnp.minimum(cnt + i * _L, _HITCAP)
        vtail = hv_v[pl.ds(s, _L)]
        hv_v[pl.ds(s, _L)] = jnp.where(lane + s >= cnt, full(_SENTINEL), vtail)
        return c

    repad(0, 0)

    nlvec = (_HITCAP + _L) // _L

    def sweep_round(r, carry):
        fetch(r + 2)
        rb = r % 2
        pltpu.make_async_copy(
            y_hbm.at[pl.ds(_BATCH, _ROWCAP)], rows2.at[rb], sems).wait()
        for _ in range(_RCOLS):
            pltpu.make_async_copy(
                wt_hbm.at[:, pl.ds(0, _TCOL)], tb.at[0], semf).wait()

        pos2[rb, pl.ds(0, _L)] = full(dump_row)
        pos2[rb, pl.ds(_L, _L)] = full(dump_row)
        lo_r = col_lo + r * _RCOLS
        hi_r = jnp.minimum(lo_r + _RCOLS, col_hi)
        sbase = (r % 3) * _RCOLS

        def rescan(t, hw):
            hvv = hv_v[pl.ds(t * _L, _L)]
            cvv = lax.shift_right_logical(hvv, 7)
            mask = (cvv >= lo_r) & (cvv < hi_r)

            def has_hits(state):
                m, _ = state
                return plsc.all_reduce_population_count(m)[0] > 0

            def take_hit(state):
                m, hw2 = state
                l = plsc.all_reduce_ffs(m)[0]
                lv = full(l)
                sel = lane == lv
                v_l = jnp.sum(jnp.where(sel, hvv, 0))
                p_l = jnp.sum(jnp.where(sel, hp_v[pl.ds(t * _L, _L)], 0))
                hw_c = jnp.minimum(hw2, _ROWCAP - 1)
                plsc.store_scatter(pos2.at[rb], [full(hw_c)], full(p_l),
                                   mask=lane0)
                c_l = lax.shift_right_logical(v_l, 7)
                slotv = full(sbase + c_l - lo_r)
                bc = full(jnp.bitwise_and(v_l, _TCOL - 1))
                for f16 in range(_MODEL_DIM // _L):
                    fvec = lane + f16 * _L
                    vals = plsc.load_gather(tb, [slotv, fvec, bc]) * _SCALE
                    rows2[rb, hw_c, pl.ds(f16 * _L, _L)] = vals
                return m & (lane != lv), hw2 + 1

            _, hw = lax.while_loop(has_hits, take_hit, (mask, hw))
            return hw

        lax.fori_loop(0, nlvec, rescan, 0)
        pltpu.async_copy(rows2.at[rb], y_hbm.at[pos2.at[rb]], sems)
        return carry

    lax.fori_loop(0, _NROUND, sweep_round, 0)

    for _ in range(2 * _RCOLS):
        pltpu.make_async_copy(
            wt_hbm.at[:, pl.ds(0, _TCOL)], tb.at[0], semf).wait()
    pltpu.make_async_copy(
        y_hbm.at[pl.ds(_BATCH, _ROWCAP)], rows2.at[0], sems).wait()
    pltpu.make_async_copy(
        y_hbm.at[pl.ds(_BATCH, _ROWCAP)], rows2.at[1], sems).wait()


def kernel(x, W):
    y = _emb_sweep(x, W.T)
    return y[:_BATCH, :_MODEL_DIM]


# speedup vs baseline: 3.7362x; 3.6724x over previous
import functools
import math

import jax
import jax.numpy as jnp
from jax import lax
from jax.experimental import pallas as pl
from jax.experimental.pallas import tpu as pltpu
from jax.experimental.pallas import tpu_sc as plsc

_MODEL_DIM = 64
_VOCAB = 1000000
_BATCH = 16384
_SCALE = math.sqrt(_MODEL_DIM)

_info = plsc.get_sparse_core_info()
_NC = _info.num_cores
_NS = _info.num_subcores
_L = _info.num_lanes
_NW = _NC * _NS
_TCOL = 128
_NCOLS = (_VOCAB + _TCOL - 1) // _TCOL
_RCOLS = 4
_RW = _RCOLS * _TCOL
_NROUND = ((_NCOLS + _NW - 1) // _NW + _RCOLS - 1) // _RCOLS
_HITCAP = 704
_ROWCAP = 32
_XCHUNK = 2048
_YROWS = _BATCH + 1024
_SENTINEL = _NCOLS * _TCOL + 1

_mesh = plsc.VectorSubcoreMesh(core_axis_name="c", subcore_axis_name="s")


@functools.partial(
    pl.kernel,
    mesh=_mesh,
    compiler_params=pltpu.CompilerParams(needs_layout_passes=False),
    out_type=jax.ShapeDtypeStruct((_YROWS, _TCOL), jnp.float32),
    scratch_types=[
        pltpu.VMEM((2, _XCHUNK), jnp.int32),
        pltpu.VMEM((3 * _RCOLS, _MODEL_DIM, _TCOL), jnp.float32),
        pltpu.VMEM((_HITCAP + _L,), jnp.int32),
        pltpu.VMEM((_HITCAP + _L,), jnp.int32),
        pltpu.VMEM((2, _ROWCAP, _TCOL), jnp.float32),
        pltpu.VMEM((2, _ROWCAP), jnp.int32),
        pltpu.VMEM((1, _TCOL), jnp.float32),
        pltpu.SemaphoreType.DMA,
        pltpu.SemaphoreType.DMA,
        pltpu.SemaphoreType.DMA,
    ],
)
def _emb_sweep(x_hbm, wt_hbm, y_hbm, xc_v, tb, hv_v, hp_v, rows2, pos2,
               junk_v, semx, semf, sems):
    wid = lax.axis_index("s") * _NC + lax.axis_index("c")
    col_lo = wid * _NCOLS // _NW
    col_hi = (wid + 1) * _NCOLS // _NW
    lane = lax.iota(jnp.int32, _L)
    lane0 = lane == 0
    dump_base = _BATCH + wid * _ROWCAP

    def full(v):
        return jnp.full((_L,), v, jnp.int32)

    def fill_sent(i, c):
        hv_v[pl.ds(i * _L, _L)] = full(_SENTINEL)
        return c

    lax.fori_loop(0, (_HITCAP + _L) // _L, fill_sent, 0)

    def fetch(r):
        base = (r % 3) * _RCOLS
        for j in range(_RCOLS):
            q = jnp.minimum(col_lo + r * _RCOLS + j, _NCOLS - 1) * _TCOL
            pltpu.async_copy(
                wt_hbm.at[:, pl.ds(pl.multiple_of(q, _TCOL), _TCOL)],
                tb.at[base + j], semf)

    for i in range(2 * _ROWCAP):
        pltpu.async_copy(y_hbm.at[pl.ds(_BATCH + i, 1)], junk_v, sems)
    fetch(0)
    fetch(1)

    first_x = pltpu.async_copy(x_hbm.at[pl.ds(0, _XCHUNK)], xc_v.at[0], semx)
    first_x.wait()

    def scan_chunk(ch, cnt):
        nxt = jnp.minimum(ch + 1, _BATCH // _XCHUNK - 1)
        cp = pltpu.async_copy(
            x_hbm.at[pl.ds(nxt * _XCHUNK, _XCHUNK)], xc_v.at[(ch + 1) % 2],
            semx)

        def scan_vec(t, cnt2):
            v = xc_v[ch % 2, pl.ds(t * _L, _L)]
            c = lax.shift_right_logical(v, 7)
            mask = (c >= col_lo) & (c < col_hi)
            pos = full(ch * _XCHUNK + t * _L) + lane
            cc = jnp.minimum(cnt2, _HITCAP)
            plsc.store_compressed(hv_v.at[pl.ds(cc, _L)], v, mask=mask)
            plsc.store_compressed(hp_v.at[pl.ds(cc, _L)], pos, mask=mask)
            return cnt2 + plsc.all_reduce_population_count(mask)[0]

        cnt = lax.fori_loop(0, _XCHUNK // _L, scan_vec, cnt)
        cp.wait()
        return cnt

    cnt = lax.fori_loop(0, _BATCH // _XCHUNK, scan_chunk, 0)
    def repad(i, c):
        s = jnp.minimum(cnt + i * _L, _HITCAP)
        vtail = hv_v[pl.ds(s, _L)]
        hv_v[pl.ds(s, _L)] = jnp.where(lane + s >= cnt, full(_SENTINEL), vtail)
        return c

    repad(0, 0)

    nlvec = (_HITCAP + _L) // _L

    def sweep_round(r, carry):
        fetch(r + 2)
        rb = r % 2
        pltpu.make_async_copy(
            y_hbm.at[pl.ds(_BATCH, _ROWCAP)], rows2.at[rb], sems).wait()
        for _ in range(_RCOLS):
            pltpu.make_async_copy(
                wt_hbm.at[:, pl.ds(0, _TCOL)], tb.at[0], semf).wait()

        pos2[rb, pl.ds(0, _L)] = full(dump_base) + lane
        pos2[rb, pl.ds(_L, _L)] = full(dump_base + _L) + lane
        lo_r = col_lo + r * _RCOLS
        hi_r = jnp.minimum(lo_r + _RCOLS, col_hi)
        sbase = (r % 3) * _RCOLS

        def rescan(t, hw):
            hvv = hv_v[pl.ds(t * _L, _L)]
            cvv = lax.shift_right_logical(hvv, 7)
            mask = (cvv >= lo_r) & (cvv < hi_r)

            def has_hits(state):
                m, _ = state
                return plsc.all_reduce_population_count(m)[0] > 0

            def take_hit(state):
                m, hw2 = state
                l = plsc.all_reduce_ffs(m)[0]
                lv = full(l)
                sel = lane == lv
                v_l = jnp.sum(jnp.where(sel, hvv, 0))
                p_l = jnp.sum(jnp.where(sel, hp_v[pl.ds(t * _L, _L)], 0))
                hw_c = jnp.minimum(hw2, _ROWCAP - 1)
                plsc.store_scatter(pos2.at[rb], [full(hw_c)], full(p_l),
                                   mask=lane0)
                c_l = lax.shift_right_logical(v_l, 7)
                slotv = full(sbase + c_l - lo_r)
                bc = full(jnp.bitwise_and(v_l, _TCOL - 1))
                for f16 in range(_MODEL_DIM // _L):
                    fvec = lane + f16 * _L
                    vals = plsc.load_gather(tb, [slotv, fvec, bc]) * _SCALE
                    rows2[rb, hw_c, pl.ds(f16 * _L, _L)] = vals
                return m & (lane != lv), hw2 + 1

            _, hw = lax.while_loop(has_hits, take_hit, (mask, hw))
            return hw

        lax.fori_loop(0, nlvec, rescan, 0)
        pltpu.async_copy(rows2.at[rb], y_hbm.at[pos2.at[rb]], sems)
        return carry

    lax.fori_loop(0, _NROUND, sweep_round, 0)

    for _ in range(2 * _RCOLS):
        pltpu.make_async_copy(
            wt_hbm.at[:, pl.ds(0, _TCOL)], tb.at[0], semf).wait()
    pltpu.make_async_copy(
        y_hbm.at[pl.ds(_BATCH, _ROWCAP)], rows2.at[0], sems).wait()
    pltpu.make_async_copy(
        y_hbm.at[pl.ds(_BATCH, _ROWCAP)], rows2.at[1], sems).wait()


def kernel(x, W):
    y = _emb_sweep(x, W.T)
    return y[:_BATCH, :_MODEL_DIM]
